# baseline (device time: 310554 ns/iter reference)
import jax
import jax.numpy as jnp
from jax import lax
from jax.experimental import pallas as pl
from jax.experimental.pallas import tpu as pltpu

N_DEV = 4
N_DIR = 2
TILES_PER_DIR = 4
N_HALF = 4


def kernel(x, w_mat, scale_x, scale_w):
    m_global, k_shard = x.shape
    _, n = w_mat.shape
    m_per = m_global // N_DEV
    ncols_dir = n // N_DIR
    nt = ncols_dir // TILES_PER_DIR
    nh = nt // N_HALF

    def body(x_ref, w_ref, sx_ref, sw_ref, out_ref,
             w_bf, send_buf, recv_buf, own_bf, stage,
             send_sems, recv_sems, copy_sems):
        d = lax.axis_index("i")
        left = lax.rem(d + N_DEV - 1, N_DEV)
        right = lax.rem(d + 1, N_DEV)

        barrier_sem = pltpu.get_barrier_semaphore()
        pl.semaphore_signal(barrier_sem, inc=1, device_id=(left,),
                            device_id_type=pl.DeviceIdType.MESH)
        pl.semaphore_signal(barrier_sem, inc=1, device_id=(right,),
                            device_id_type=pl.DeviceIdType.MESH)
        pl.semaphore_wait(barrier_sem, 2)

        target = {0: right, 1: left}

        def chunk_at_hop(dir_, s):
            if dir_ == 0:
                return lax.rem(d + 2 * N_DEV - 1 - s, N_DEV)
            return lax.rem(d + 1 + s, N_DEV)

        def col0(dir_, t):
            return dir_ * ncols_dir + t * nt

        def load_w_tile(dir_, t):
            w_bf[dir_] = w_ref[:, pl.ds(col0(dir_, t), nt)].astype(
                jnp.bfloat16)

        def partial_half(dir_, c, h):
            xc = x_ref[pl.ds(c * m_per, m_per), :].astype(jnp.bfloat16)
            ph = jnp.dot(xc, w_bf[dir_, :, pl.ds(h * nh, nh)],
                         preferred_element_type=jnp.float32)
            return ph.astype(jnp.bfloat16)

        def half_rdma(dir_, j, h):
            return pltpu.make_async_remote_copy(
                src_ref=send_buf.at[dir_, j % 2, :, pl.ds(h * nh, nh)],
                dst_ref=recv_buf.at[dir_, j % 2, :, pl.ds(h * nh, nh)],
                send_sem=send_sems.at[dir_, j % 2, h],
                recv_sem=recv_sems.at[dir_, j % 2, h],
                device_id=(target[dir_],),
                device_id_type=pl.DeviceIdType.MESH,
            )

        rdmas = {}
        copies = {}

        def issue_hop0(dir_, j):
            c = chunk_at_hop(dir_, 0)
            for h in range(N_HALF):
                if j >= 2:
                    rdmas[dir_, j - 2, h].wait_send()
                send_buf[dir_, j % 2, :, pl.ds(h * nh, nh)] = \
                    partial_half(dir_, c, h)
                r = half_rdma(dir_, j, h)
                r.start()
                rdmas[dir_, j, h] = r

        for dir_ in range(N_DIR):
            load_w_tile(dir_, 0)
            issue_hop0(dir_, 0)

        for t in range(TILES_PER_DIR):
            for s in (1, 2):
                j = 3 * t + s
                for dir_ in range(N_DIR):
                    c = chunk_at_hop(dir_, s)
                    for h in range(N_HALF):
                        if j >= 2:
                            rdmas[dir_, j - 2, h].wait_send()
                        send_buf[dir_, j % 2, :, pl.ds(h * nh, nh)] = \
                            partial_half(dir_, c, h)
                for h in range(N_HALF):
                    for dir_ in range(N_DIR):
                        hs = pl.ds(h * nh, nh)
                        rdmas[dir_, j - 1, h].wait_recv()
                        send_buf[dir_, j % 2, :, hs] = (
                            send_buf[dir_, j % 2, :, hs]
                            + recv_buf[dir_, (j - 1) % 2, :, hs])
                        r = half_rdma(dir_, j, h)
                        r.start()
                        rdmas[dir_, j, h] = r

            for dir_ in range(N_DIR):
                for h in range(N_HALF):
                    own_bf[dir_, :, pl.ds(h * nh, nh)] = \
                        partial_half(dir_, d, h)

            if t < TILES_PER_DIR - 1:
                for dir_ in range(N_DIR):
                    load_w_tile(dir_, t + 1)
                    issue_hop0(dir_, 3 * (t + 1))

            jf = 3 * t + 2
            scale = sx_ref[0, 0] * sw_ref[0, 0]
            for dir_ in range(N_DIR):
                if t > 0:
                    copies[dir_, t - 1].wait()
                for h in range(N_HALF):
                    hs = pl.ds(h * nh, nh)
                    rdmas[dir_, jf, h].wait_recv()
                    acc = own_bf[dir_, :, hs] + recv_buf[dir_, jf % 2, :, hs]
                    stage[dir_, :, hs] = jnp.maximum(
                        acc.astype(jnp.float32) * scale, 0.0)
                cp = pltpu.make_async_copy(
                    stage.at[dir_],
                    out_ref.at[:, pl.ds(col0(dir_, t), nt)],
                    copy_sems.at[dir_])
                cp.start()
                copies[dir_, t] = cp

        for dir_ in range(N_DIR):
            for j in (3 * TILES_PER_DIR - 2, 3 * TILES_PER_DIR - 1):
                for h in range(N_HALF):
                    rdmas[dir_, j, h].wait_send()
            copies[dir_, TILES_PER_DIR - 1].wait()

    return pl.pallas_call(
        body,
        in_specs=[
            pl.BlockSpec(memory_space=pltpu.VMEM),
            pl.BlockSpec(memory_space=pltpu.VMEM),
            pl.BlockSpec(memory_space=pltpu.SMEM),
            pl.BlockSpec(memory_space=pltpu.SMEM),
        ],
        out_specs=pl.BlockSpec(memory_space=pl.ANY),
        out_shape=jax.ShapeDtypeStruct((m_per, n), jnp.float32),
        scratch_shapes=[
            pltpu.VMEM((N_DIR, k_shard, nt), jnp.bfloat16),
            pltpu.VMEM((N_DIR, 2, m_per, nt), jnp.bfloat16),
            pltpu.VMEM((N_DIR, 2, m_per, nt), jnp.bfloat16),
            pltpu.VMEM((N_DIR, m_per, nt), jnp.bfloat16),
            pltpu.VMEM((N_DIR, m_per, nt), jnp.float32),
            pltpu.SemaphoreType.DMA((N_DIR, 2, N_HALF)),
            pltpu.SemaphoreType.DMA((N_DIR, 2, N_HALF)),
            pltpu.SemaphoreType.DMA((N_DIR,)),
        ],
        compiler_params=pltpu.CompilerParams(
            collective_id=0,
            vmem_limit_bytes=52 * 1024 * 1024,
        ),
    )(x, w_mat, scale_x.reshape(1, 1), scale_w.reshape(1, 1))


# device time: 308563 ns/iter; 1.0065x vs baseline; 1.0065x over previous
import jax
import jax.numpy as jnp
from jax import lax
from jax.experimental import pallas as pl
from jax.experimental.pallas import tpu as pltpu

N_DEV = 4
N_DIR = 2
TILES_PER_DIR = 4
N_HALF = 2


def kernel(x, w_mat, scale_x, scale_w):
    m_global, k_shard = x.shape
    _, n = w_mat.shape
    m_per = m_global // N_DEV
    ncols_dir = n // N_DIR
    nt = ncols_dir // TILES_PER_DIR
    nh = nt // N_HALF

    n_hops = 3 * TILES_PER_DIR

    def body(x_ref, w_ref, sx_ref, sw_ref, out_ref,
             w_bf, send_buf, recv_buf, own_bf, stage,
             send_sems, recv_sems, copy_sems, credit_sems):
        d = lax.axis_index("i")
        left = lax.rem(d + N_DEV - 1, N_DEV)
        right = lax.rem(d + 1, N_DEV)

        barrier_sem = pltpu.get_barrier_semaphore()
        pl.semaphore_signal(barrier_sem, inc=1, device_id=(left,),
                            device_id_type=pl.DeviceIdType.MESH)
        pl.semaphore_signal(barrier_sem, inc=1, device_id=(right,),
                            device_id_type=pl.DeviceIdType.MESH)
        pl.semaphore_wait(barrier_sem, 2)

        target = {0: right, 1: left}

        def chunk_at_hop(dir_, s):
            if dir_ == 0:
                return lax.rem(d + 2 * N_DEV - 1 - s, N_DEV)
            return lax.rem(d + 1 + s, N_DEV)

        def col0(dir_, t):
            return dir_ * ncols_dir + t * nt

        def load_w_tile(dir_, t):
            w_bf[dir_] = w_ref[:, pl.ds(col0(dir_, t), nt)].astype(
                jnp.bfloat16)

        def partial_half(dir_, c, h):
            xc = x_ref[pl.ds(c * m_per, m_per), :].astype(jnp.bfloat16)
            ph = jnp.dot(xc, w_bf[dir_, :, pl.ds(h * nh, nh)],
                         preferred_element_type=jnp.float32)
            return ph.astype(jnp.bfloat16)

        def half_rdma(dir_, j, h):
            return pltpu.make_async_remote_copy(
                src_ref=send_buf.at[dir_, j % 2, :, pl.ds(h * nh, nh)],
                dst_ref=recv_buf.at[dir_, j % 2, :, pl.ds(h * nh, nh)],
                send_sem=send_sems.at[dir_, j % 2, h],
                recv_sem=recv_sems.at[dir_, j % 2, h],
                device_id=(target[dir_],),
                device_id_type=pl.DeviceIdType.MESH,
            )

        rdmas = {}
        copies = {}

        def consumed(dir_, j, h):
            if j < n_hops - 2:
                pl.semaphore_signal(
                    credit_sems.at[dir_, j % 2, h], inc=1,
                    device_id=(target[1 - dir_],),
                    device_id_type=pl.DeviceIdType.MESH)

        def await_credit(dir_, j, h):
            if j >= 2:
                pl.semaphore_wait(credit_sems.at[dir_, j % 2, h], 1)

        def issue_hop0(dir_, j):
            c = chunk_at_hop(dir_, 0)
            for h in range(N_HALF):
                if j >= 2:
                    rdmas[dir_, j - 2, h].wait_send()
                send_buf[dir_, j % 2, :, pl.ds(h * nh, nh)] = \
                    partial_half(dir_, c, h)
                await_credit(dir_, j, h)
                r = half_rdma(dir_, j, h)
                r.start()
                rdmas[dir_, j, h] = r

        for dir_ in range(N_DIR):
            load_w_tile(dir_, 0)
            issue_hop0(dir_, 0)

        for t in range(TILES_PER_DIR):
            for s in (1, 2):
                j = 3 * t + s
                for dir_ in range(N_DIR):
                    c = chunk_at_hop(dir_, s)
                    for h in range(N_HALF):
                        if j >= 2:
                            rdmas[dir_, j - 2, h].wait_send()
                        send_buf[dir_, j % 2, :, pl.ds(h * nh, nh)] = \
                            partial_half(dir_, c, h)
                for h in range(N_HALF):
                    for dir_ in range(N_DIR):
                        hs = pl.ds(h * nh, nh)
                        rdmas[dir_, j - 1, h].wait_recv()
                        send_buf[dir_, j % 2, :, hs] = (
                            send_buf[dir_, j % 2, :, hs]
                            + recv_buf[dir_, (j - 1) % 2, :, hs])
                        consumed(dir_, j - 1, h)
                        await_credit(dir_, j, h)
                        r = half_rdma(dir_, j, h)
                        r.start()
                        rdmas[dir_, j, h] = r

            for dir_ in range(N_DIR):
                for h in range(N_HALF):
                    own_bf[dir_, :, pl.ds(h * nh, nh)] = \
                        partial_half(dir_, d, h)

            if t < TILES_PER_DIR - 1:
                for dir_ in range(N_DIR):
                    load_w_tile(dir_, t + 1)
                    issue_hop0(dir_, 3 * (t + 1))

            jf = 3 * t + 2
            scale = sx_ref[0, 0] * sw_ref[0, 0]
            for dir_ in range(N_DIR):
                if t > 0:
                    copies[dir_, t - 1].wait()
                for h in range(N_HALF):
                    hs = pl.ds(h * nh, nh)
                    rdmas[dir_, jf, h].wait_recv()
                    acc = own_bf[dir_, :, hs] + recv_buf[dir_, jf % 2, :, hs]
                    stage[dir_, :, hs] = jnp.maximum(
                        acc.astype(jnp.float32) * scale, 0.0)
                    consumed(dir_, jf, h)
                cp = pltpu.make_async_copy(
                    stage.at[dir_],
                    out_ref.at[:, pl.ds(col0(dir_, t), nt)],
                    copy_sems.at[dir_])
                cp.start()
                copies[dir_, t] = cp

        for dir_ in range(N_DIR):
            for j in (3 * TILES_PER_DIR - 2, 3 * TILES_PER_DIR - 1):
                for h in range(N_HALF):
                    rdmas[dir_, j, h].wait_send()
            copies[dir_, TILES_PER_DIR - 1].wait()

    return pl.pallas_call(
        body,
        in_specs=[
            pl.BlockSpec(memory_space=pltpu.VMEM),
            pl.BlockSpec(memory_space=pltpu.VMEM),
            pl.BlockSpec(memory_space=pltpu.SMEM),
            pl.BlockSpec(memory_space=pltpu.SMEM),
        ],
        out_specs=pl.BlockSpec(memory_space=pl.ANY),
        out_shape=jax.ShapeDtypeStruct((m_per, n), jnp.float32),
        scratch_shapes=[
            pltpu.VMEM((N_DIR, k_shard, nt), jnp.bfloat16),
            pltpu.VMEM((N_DIR, 2, m_per, nt), jnp.bfloat16),
            pltpu.VMEM((N_DIR, 2, m_per, nt), jnp.bfloat16),
            pltpu.VMEM((N_DIR, m_per, nt), jnp.bfloat16),
            pltpu.VMEM((N_DIR, m_per, nt), jnp.float32),
            pltpu.SemaphoreType.DMA((N_DIR, 2, N_HALF)),
            pltpu.SemaphoreType.DMA((N_DIR, 2, N_HALF)),
            pltpu.SemaphoreType.DMA((N_DIR,)),
            pltpu.SemaphoreType.REGULAR((N_DIR, 2, N_HALF)),
        ],
        compiler_params=pltpu.CompilerParams(
            collective_id=0,
            vmem_limit_bytes=52 * 1024 * 1024,
        ),
    )(x, w_mat, scale_x.reshape(1, 1), scale_w.reshape(1, 1))
